# baseline (speedup 1.0000x reference)
import functools

import jax
import jax.numpy as jnp
from jax import lax
from jax.experimental import pallas as pl
from jax.experimental.pallas import tpu as pltpu

_BLK_K = 6144


def _mm_kernel(bt_ref, at_ref, o_ref, *, nsteps, tail):
    k = pl.program_id(0)

    @pl.when(k == 0)
    def _init():
        o_ref[...] = jnp.zeros_like(o_ref)

    @pl.when(k < nsteps - 1)
    def _body():
        o_ref[...] += jnp.dot(
            bt_ref[...].astype(jnp.bfloat16),
            at_ref[...].astype(jnp.bfloat16),
            preferred_element_type=jnp.float32,
        )

    @pl.when(k == nsteps - 1)
    def _tail():
        bt = bt_ref[...]
        col = lax.broadcasted_iota(jnp.int32, bt.shape, 1)
        bt = jnp.where(col < tail, bt, 0.0)
        at = at_ref[...]
        row = lax.broadcasted_iota(jnp.int32, at.shape, 0)
        at = jnp.where(row < tail, at, 0.0)
        o_ref[...] += jnp.dot(
            bt.astype(jnp.bfloat16),
            at.astype(jnp.bfloat16),
            preferred_element_type=jnp.float32,
        )


def kernel(attention_weights, content_matrix):
    m, k_dim = attention_weights.shape
    _, n = content_matrix.shape
    at = attention_weights.T
    bt = content_matrix.T
    nsteps = pl.cdiv(k_dim, _BLK_K)
    tail = k_dim - (nsteps - 1) * _BLK_K
    body = functools.partial(_mm_kernel, nsteps=nsteps, tail=tail)
    out_t = pl.pallas_call(
        body,
        grid=(nsteps,),
        in_specs=[
            pl.BlockSpec((n, _BLK_K), lambda k: (0, k)),
            pl.BlockSpec((_BLK_K, m), lambda k: (k, 0)),
        ],
        out_specs=pl.BlockSpec((n, m), lambda k: (0, 0)),
        out_shape=jax.ShapeDtypeStruct((n, m), jnp.float32),
        compiler_params=pltpu.CompilerParams(
            dimension_semantics=("arbitrary",)
        ),
    )(bt, at)
    return out_t.T


# speedup vs baseline: 1.0431x; 1.0431x over previous
import functools

import jax
import jax.numpy as jnp
from jax import lax
from jax.experimental import pallas as pl
from jax.experimental.pallas import tpu as pltpu

_BLK_K = 3072


def _mm_kernel(bt_ref, at_ref, o_ref, *, nsteps, tail):
    k = pl.program_id(0)

    @pl.when(k == 0)
    def _init():
        o_ref[...] = jnp.zeros_like(o_ref)

    @pl.when(k < nsteps - 1)
    def _body():
        o_ref[...] += jnp.dot(
            bt_ref[...].astype(jnp.bfloat16),
            at_ref[...].astype(jnp.bfloat16),
            preferred_element_type=jnp.float32,
        )

    @pl.when(k == nsteps - 1)
    def _tail():
        bt = bt_ref[...]
        col = lax.broadcasted_iota(jnp.int32, bt.shape, 1)
        bt = jnp.where(col < tail, bt, 0.0)
        at = at_ref[...]
        row = lax.broadcasted_iota(jnp.int32, at.shape, 0)
        at = jnp.where(row < tail, at, 0.0)
        o_ref[...] += jnp.dot(
            bt.astype(jnp.bfloat16),
            at.astype(jnp.bfloat16),
            preferred_element_type=jnp.float32,
        )


def kernel(attention_weights, content_matrix):
    m, k_dim = attention_weights.shape
    _, n = content_matrix.shape
    at = attention_weights.T
    bt = content_matrix.T
    nsteps = pl.cdiv(k_dim, _BLK_K)
    tail = k_dim - (nsteps - 1) * _BLK_K
    body = functools.partial(_mm_kernel, nsteps=nsteps, tail=tail)
    out_t = pl.pallas_call(
        body,
        grid=(nsteps,),
        in_specs=[
            pl.BlockSpec((n, _BLK_K), lambda k: (0, k)),
            pl.BlockSpec((_BLK_K, m), lambda k: (k, 0)),
        ],
        out_specs=pl.BlockSpec((n, m), lambda k: (0, 0)),
        out_shape=jax.ShapeDtypeStruct((n, m), jnp.float32),
        compiler_params=pltpu.CompilerParams(
            dimension_semantics=("arbitrary",)
        ),
    )(bt, at)
    return out_t.T
